# baseline (speedup 1.0000x reference)
import jax
import jax.numpy as jnp
from jax import lax
from jax.experimental import pallas as pl
from jax.experimental.pallas import tpu as pltpu
from jax.experimental.pallas import tpu_sc as plsc

NUM_EMBEDDINGS = 1000000
EMBEDDING_DIM = 32
BATCH = 16384
N_FIELDS = 26

NUM_LOOKUPS = BATCH * N_FIELDS
NC, NS = 2, 16
NW = NC * NS
B_PER_W = NUM_LOOKUPS // NW
N_CHUNKS = 8
CHUNK = B_PER_W // N_CHUNKS

BB_PER_W = BATCH // (128 * NW)
QROWS = 32
QLOOK = QROWS * N_FIELDS
NQ = B_PER_W // QLOOK
RPAD = EMBEDDING_DIM + 1


def _gather_sc(idx_hbm, table_hbm, out_hbm, idx_v, rows0, rows1, gsem0, gsem1,
               wsem0, wsem1):
    wid = lax.axis_index("s") * NC + lax.axis_index("c")
    base = wid * B_PER_W
    rows = (rows0, rows1)
    gsem = (gsem0, gsem1)
    wsem = (wsem0, wsem1)

    pltpu.sync_copy(idx_hbm.at[wid], idx_v)

    def gather(c):
        return pltpu.make_async_copy(
            table_hbm.at[idx_v.at[pl.ds(c * CHUNK, CHUNK)]],
            rows[c % 2], gsem[c % 2])

    def writeback(c):
        return pltpu.make_async_copy(
            rows[c % 2],
            out_hbm.at[pl.ds(base + c * CHUNK, CHUNK)],
            wsem[c % 2])

    gather(0).start()
    for c in range(N_CHUNKS):
        gather(c).wait()
        if c + 1 < N_CHUNKS:
            if c >= 1:
                writeback(c - 1).wait()
            gather(c + 1).start()
        writeback(c).start()
    writeback(N_CHUNKS - 2).wait()
    writeback(N_CHUNKS - 1).wait()


def _format_sc(rows_hbm, out5_hbm, r0, r1, t0, t1, lsem0, lsem1,
               wsem0, wsem1):
    wid = lax.axis_index("s") * NC + lax.axis_index("c")
    base = wid * B_PER_W
    r_v = (r0, r1)
    t_v = (t0, t1)
    lsem = (lsem0, lsem1)
    wsem = (wsem0, wsem1)

    def load(q, p):
        return pltpu.make_async_copy(
            rows_hbm.at[pl.ds(base + q * QLOOK, QLOOK)],
            r_v[p].at[:, pl.ds(0, EMBEDDING_DIM)], lsem[p])

    def writeback(q, p):
        bblk = wid * BB_PER_W + lax.shift_right_logical(q, 2)
        boff = lax.bitwise_and(q, 3) * QROWS
        return pltpu.make_async_copy(
            t_v[p], out5_hbm.at[:, :, bblk, :, pl.ds(boff, QROWS)],
            wsem[p])

    def retile(p):
        rr, tt = r_v[p], t_v[p]

        def per_field(f, _):
            iota = lax.iota(jnp.int32, 16)
            for g in range(QROWS // 16):
                row_idx = iota * N_FIELDS + (g * 16 * N_FIELDS + f)
                for d in range(EMBEDDING_DIM):
                    col_idx = jnp.full((16,), d, jnp.int32)
                    vals = plsc.load_gather(rr, [row_idx, col_idx])
                    tt[f, d // 8, d % 8, pl.ds(g * 16, 16)] = vals
            return 0

        lax.fori_loop(0, N_FIELDS, per_field, 0)

    load(0, 0).start()

    def two_steps(qq, _):
        for par in (0, 1):
            q = 2 * qq + par
            load(q, par).wait()

            @pl.when(qq >= 1)
            def _():
                writeback(q - 2, par).wait()

            if par == 0:
                load(q + 1, 1).start()
            else:
                @pl.when(qq < NQ // 2 - 1)
                def _():
                    load(q + 1, 0).start()

            retile(par)
            writeback(q, par).start()
        return 0

    lax.fori_loop(0, NQ // 2, two_steps, 0)
    writeback(NQ - 2, 0).wait()
    writeback(NQ - 1, 1).wait()


@jax.jit
def kernel(token_ids, weight):
    idx = token_ids.reshape(NW, B_PER_W)
    mesh = plsc.VectorSubcoreMesh(core_axis_name="c", subcore_axis_name="s")
    rows = pl.kernel(
        _gather_sc,
        mesh=mesh,
        out_type=jax.ShapeDtypeStruct((NUM_LOOKUPS, EMBEDDING_DIM),
                                      jnp.float32),
        scratch_types=[
            pltpu.VMEM((B_PER_W,), jnp.int32),
            pltpu.VMEM((CHUNK, EMBEDDING_DIM), jnp.float32),
            pltpu.VMEM((CHUNK, EMBEDDING_DIM), jnp.float32),
            pltpu.SemaphoreType.DMA,
            pltpu.SemaphoreType.DMA,
            pltpu.SemaphoreType.DMA,
            pltpu.SemaphoreType.DMA,
        ],
        compiler_params=pltpu.CompilerParams(use_tc_tiling_on_sc=False),
    )(idx, weight)
    out5 = pl.kernel(
        _format_sc,
        mesh=mesh,
        out_type=jax.ShapeDtypeStruct(
            (N_FIELDS, EMBEDDING_DIM // 8, BATCH // 128, 8, 128), jnp.float32),
        scratch_types=[
            pltpu.VMEM((QLOOK, RPAD), jnp.float32),
            pltpu.VMEM((QLOOK, RPAD), jnp.float32),
            pltpu.VMEM((N_FIELDS, EMBEDDING_DIM // 8, 8, QROWS), jnp.float32),
            pltpu.VMEM((N_FIELDS, EMBEDDING_DIM // 8, 8, QROWS), jnp.float32),
            pltpu.SemaphoreType.DMA,
            pltpu.SemaphoreType.DMA,
            pltpu.SemaphoreType.DMA,
            pltpu.SemaphoreType.DMA,
        ],
        compiler_params=pltpu.CompilerParams(use_tc_tiling_on_sc=False,
                                             needs_layout_passes=False),
    )(rows)
    return out5.transpose(2, 4, 0, 1, 3).reshape(BATCH, N_FIELDS,
                                                 EMBEDDING_DIM)


# speedup vs baseline: 1.2078x; 1.2078x over previous
import jax
import jax.numpy as jnp
from jax import lax
from jax.experimental import pallas as pl
from jax.experimental.pallas import tpu as pltpu
from jax.experimental.pallas import tpu_sc as plsc

NUM_EMBEDDINGS = 1000000
EMBEDDING_DIM = 32
BATCH = 16384
N_FIELDS = 26

NUM_LOOKUPS = BATCH * N_FIELDS
NC, NS = 2, 16
NW = NC * NS
B_PER_W = NUM_LOOKUPS // NW
N_CHUNKS = 8
CHUNK = B_PER_W // N_CHUNKS

BB_PER_W = BATCH // (128 * NW)
QROWS = 32
QLOOK = QROWS * N_FIELDS
NQ = B_PER_W // QLOOK
RPAD = EMBEDDING_DIM + 1


def _gather_sc(idx_hbm, table_hbm, out_hbm, idx_v, rows0, rows1, gsem0, gsem1,
               wsem0, wsem1):
    wid = lax.axis_index("s") * NC + lax.axis_index("c")
    base = wid * B_PER_W
    rows = (rows0, rows1)
    gsem = (gsem0, gsem1)
    wsem = (wsem0, wsem1)

    pltpu.sync_copy(idx_hbm.at[wid], idx_v)

    def gather(c):
        return pltpu.make_async_copy(
            table_hbm.at[idx_v.at[pl.ds(c * CHUNK, CHUNK)]],
            rows[c % 2], gsem[c % 2])

    def writeback(c):
        return pltpu.make_async_copy(
            rows[c % 2],
            out_hbm.at[pl.ds(base + c * CHUNK, CHUNK)],
            wsem[c % 2])

    gather(0).start()
    for c in range(N_CHUNKS):
        gather(c).wait()
        if c + 1 < N_CHUNKS:
            if c >= 1:
                writeback(c - 1).wait()
            gather(c + 1).start()
        writeback(c).start()
    writeback(N_CHUNKS - 2).wait()
    writeback(N_CHUNKS - 1).wait()


def _format_sc(rows_hbm, out5_hbm, r0, r1, t0, t1, lsem0, lsem1,
               wsem0, wsem1):
    wid = lax.axis_index("s") * NC + lax.axis_index("c")
    base = wid * B_PER_W
    r_v = (r0, r1)
    t_v = (t0, t1)
    lsem = (lsem0, lsem1)
    wsem = (wsem0, wsem1)

    def load(q, p):
        return pltpu.make_async_copy(
            rows_hbm.at[pl.ds(base + q * QLOOK, QLOOK)],
            r_v[p].at[:, pl.ds(0, EMBEDDING_DIM)], lsem[p])

    def writeback(q, p):
        bblk = wid * BB_PER_W + lax.shift_right_logical(q, 2)
        boff = lax.bitwise_and(q, 3) * QROWS
        return pltpu.make_async_copy(
            t_v[p], out5_hbm.at[:, :, bblk, :, pl.ds(boff, QROWS)],
            wsem[p])

    def retile(p):
        rr, tt = r_v[p], t_v[p]

        @plsc.parallel_loop(0, N_FIELDS, unroll=2)
        def per_field(f):
            iota = lax.iota(jnp.int32, 16)
            for g in range(QROWS // 16):
                row_idx = iota * N_FIELDS + (g * 16 * N_FIELDS + f)
                for d in range(EMBEDDING_DIM):
                    col_idx = jnp.full((16,), d, jnp.int32)
                    vals = plsc.load_gather(rr, [row_idx, col_idx])
                    tt[f, d // 8, d % 8, pl.ds(g * 16, 16)] = vals

    load(0, 0).start()

    def two_steps(qq, _):
        for par in (0, 1):
            q = 2 * qq + par
            load(q, par).wait()

            @pl.when(qq >= 1)
            def _():
                writeback(q - 2, par).wait()

            if par == 0:
                load(q + 1, 1).start()
            else:
                @pl.when(qq < NQ // 2 - 1)
                def _():
                    load(q + 1, 0).start()

            retile(par)
            writeback(q, par).start()
        return 0

    lax.fori_loop(0, NQ // 2, two_steps, 0)
    writeback(NQ - 2, 0).wait()
    writeback(NQ - 1, 1).wait()


@jax.jit
def kernel(token_ids, weight):
    idx = token_ids.reshape(NW, B_PER_W)
    mesh = plsc.VectorSubcoreMesh(core_axis_name="c", subcore_axis_name="s")
    rows = pl.kernel(
        _gather_sc,
        mesh=mesh,
        out_type=jax.ShapeDtypeStruct((NUM_LOOKUPS, EMBEDDING_DIM),
                                      jnp.float32),
        scratch_types=[
            pltpu.VMEM((B_PER_W,), jnp.int32),
            pltpu.VMEM((CHUNK, EMBEDDING_DIM), jnp.float32),
            pltpu.VMEM((CHUNK, EMBEDDING_DIM), jnp.float32),
            pltpu.SemaphoreType.DMA,
            pltpu.SemaphoreType.DMA,
            pltpu.SemaphoreType.DMA,
            pltpu.SemaphoreType.DMA,
        ],
        compiler_params=pltpu.CompilerParams(use_tc_tiling_on_sc=False),
    )(idx, weight)
    out5 = pl.kernel(
        _format_sc,
        mesh=mesh,
        out_type=jax.ShapeDtypeStruct(
            (N_FIELDS, EMBEDDING_DIM // 8, BATCH // 128, 8, 128), jnp.float32),
        scratch_types=[
            pltpu.VMEM((QLOOK, RPAD), jnp.float32),
            pltpu.VMEM((QLOOK, RPAD), jnp.float32),
            pltpu.VMEM((N_FIELDS, EMBEDDING_DIM // 8, 8, QROWS), jnp.float32),
            pltpu.VMEM((N_FIELDS, EMBEDDING_DIM // 8, 8, QROWS), jnp.float32),
            pltpu.SemaphoreType.DMA,
            pltpu.SemaphoreType.DMA,
            pltpu.SemaphoreType.DMA,
            pltpu.SemaphoreType.DMA,
        ],
        compiler_params=pltpu.CompilerParams(use_tc_tiling_on_sc=False,
                                             needs_layout_passes=False),
    )(rows)
    return out5.transpose(2, 4, 0, 1, 3).reshape(BATCH, N_FIELDS,
                                                 EMBEDDING_DIM)
